# baseline (speedup 1.0000x reference)
import functools

import jax
import jax.numpy as jnp
from jax.experimental import pallas as pl
from jax.experimental.pallas import tpu as pltpu

N_COMP = 2
N_NBRS = 32
BR = 256


def _d2_kernel(xb_ref, xa_ref, o_ref):
    xb = xb_ref[...]
    xa = xa_ref[...]
    g = jax.lax.dot_general(
        xb, xa, (((1,), (1,)), ((), ())),
        preferred_element_type=jnp.float32)
    sqb = jnp.sum(xb * xb, axis=1, keepdims=True)
    sqa = jnp.sum(xa * xa, axis=1)[None, :]
    o_ref[...] = sqb - 2.0 * g + sqa


def _pairwise_d2(x):
    n, f = x.shape
    return pl.pallas_call(
        _d2_kernel,
        grid=(n // BR,),
        in_specs=[
            pl.BlockSpec((BR, f), lambda i: (i, 0)),
            pl.BlockSpec((n, f), lambda i: (0, 0)),
        ],
        out_specs=pl.BlockSpec((BR, n), lambda i: (i, 0)),
        out_shape=jax.ShapeDtypeStruct((n, n), jnp.float32),
    )(x, x)


def _eval_basis(theta, d):
    p = d * (d + 1) // 2
    cols = [theta[..., i] for i in range(d)]
    for i in range(d):
        for j in range(i, d):
            cols.append(theta[..., i] * theta[..., j])
    return jnp.stack(cols[:p], axis=-1)


def kernel(x):
    n, f = x.shape
    d = N_COMP
    k = N_NBRS

    d2 = _pairwise_d2(x)
    _, idx = jax.lax.top_k(-d2, k + 1)
    idx = idx[:, 1:]

    nbrs = x[idx]
    centered = nbrs - x[:, None, :]
    cov = jnp.einsum('nkf,nkg->nfg', centered, centered) / (k - 1)
    w, v = jnp.linalg.eigh(cov)
    w_desc = w[:, ::-1]
    v_desc = v[:, :, ::-1]
    valid = (w_desc[:, :d] > 1e-10).astype(x.dtype)
    frames = jnp.swapaxes(v_desc[:, :, :d], 1, 2) * valid[:, :, None]

    theta = jnp.einsum('nkf,ndf->nkd', nbrs, frames)
    B = _eval_basis(theta, d)
    Q, _ = jnp.linalg.qr(B)
    P = jnp.eye(k, dtype=x.dtype)[None, :, :] - jnp.einsum('nkp,nlp->nkl', Q, Q)
    rows = idx[:, :, None]
    cols = idx[:, None, :]
    H = jnp.zeros((n, n), dtype=x.dtype).at[rows, cols].add(P)

    w, v = jnp.linalg.eigh(H)
    order = jnp.argsort(w)[d + 1: 2 * d + 1]
    return v[:, order]


# speedup vs baseline: 1.0072x; 1.0072x over previous
import functools

import jax
import jax.numpy as jnp
from jax.experimental import pallas as pl
from jax.experimental.pallas import tpu as pltpu

N_COMP = 2
N_NBRS = 32
BR = 256


def _knn_kernel(xb_ref, xa_ref, idx_ref, d2_ref):
    xb = xb_ref[...]
    xa = xa_ref[...]
    n = xa.shape[0]
    g = jax.lax.dot_general(
        xb, xa, (((1,), (1,)), ((), ())),
        preferred_element_type=jnp.float32)
    sqb = jnp.sum(xb * xb, axis=1, keepdims=True)
    sqa = jnp.sum(xa * xa, axis=1)[None, :]
    d2_ref[...] = sqb - 2.0 * g + sqa

    iota = jax.lax.broadcasted_iota(jnp.int32, (BR, n), 1)
    idx_ref[...] = jnp.zeros((BR, 128), jnp.int32)
    for j in range(N_NBRS + 1):
        d2b = d2_ref[...]
        m = jnp.min(d2b, axis=1, keepdims=True)
        sel = jnp.where(d2b == m, iota, n)
        jm = jnp.min(sel, axis=1, keepdims=True)
        idx_ref[:, j:j + 1] = jm
        d2_ref[...] = jnp.where(iota == jm, jnp.inf, d2b)


def _knn_idx(x):
    n, f = x.shape
    idx = pl.pallas_call(
        _knn_kernel,
        grid=(n // BR,),
        in_specs=[
            pl.BlockSpec((BR, f), lambda i: (i, 0)),
            pl.BlockSpec((n, f), lambda i: (0, 0)),
        ],
        out_specs=pl.BlockSpec((BR, 128), lambda i: (i, 0)),
        out_shape=jax.ShapeDtypeStruct((n, 128), jnp.int32),
        scratch_shapes=[pltpu.VMEM((BR, n), jnp.float32)],
    )(x, x)
    return idx[:, 1:N_NBRS + 1]


def _eval_basis(theta, d):
    p = d * (d + 1) // 2
    cols = [theta[..., i] for i in range(d)]
    for i in range(d):
        for j in range(i, d):
            cols.append(theta[..., i] * theta[..., j])
    return jnp.stack(cols[:p], axis=-1)


def kernel(x):
    n, f = x.shape
    d = N_COMP
    k = N_NBRS

    idx = _knn_idx(x)

    nbrs = x[idx]
    centered = nbrs - x[:, None, :]
    cov = jnp.einsum('nkf,nkg->nfg', centered, centered) / (k - 1)
    w, v = jnp.linalg.eigh(cov)
    w_desc = w[:, ::-1]
    v_desc = v[:, :, ::-1]
    valid = (w_desc[:, :d] > 1e-10).astype(x.dtype)
    frames = jnp.swapaxes(v_desc[:, :, :d], 1, 2) * valid[:, :, None]

    theta = jnp.einsum('nkf,ndf->nkd', nbrs, frames)
    B = _eval_basis(theta, d)
    Q, _ = jnp.linalg.qr(B)
    P = jnp.eye(k, dtype=x.dtype)[None, :, :] - jnp.einsum('nkp,nlp->nkl', Q, Q)
    rows = idx[:, :, None]
    cols = idx[:, None, :]
    H = jnp.zeros((n, n), dtype=x.dtype).at[rows, cols].add(P)

    w, v = jnp.linalg.eigh(H)
    order = jnp.argsort(w)[d + 1: 2 * d + 1]
    return v[:, order]


# speedup vs baseline: 1.0091x; 1.0018x over previous
import functools

import jax
import jax.numpy as jnp
from jax import lax
from jax.experimental import pallas as pl
from jax.experimental.pallas import tpu as pltpu
from jax.experimental.pallas import tpu_sc as plsc

N_COMP = 2
N_NBRS = 32
BR = 256


def _knn_kernel(xb_ref, xa_ref, idx_ref, d2_ref):
    xb = xb_ref[...]
    xa = xa_ref[...]
    n = xa.shape[0]
    g = jax.lax.dot_general(
        xb, xa, (((1,), (1,)), ((), ())),
        preferred_element_type=jnp.float32)
    sqb = jnp.sum(xb * xb, axis=1, keepdims=True)
    sqa = jnp.sum(xa * xa, axis=1)[None, :]
    d2_ref[...] = sqb - 2.0 * g + sqa

    iota = jax.lax.broadcasted_iota(jnp.int32, (BR, n), 1)
    idx_ref[...] = jnp.zeros((BR, 128), jnp.int32)
    for j in range(N_NBRS + 1):
        d2b = d2_ref[...]
        m = jnp.min(d2b, axis=1, keepdims=True)
        sel = jnp.where(d2b == m, iota, n)
        jm = jnp.min(sel, axis=1, keepdims=True)
        idx_ref[:, j:j + 1] = jm
        d2_ref[...] = jnp.where(iota == jm, jnp.inf, d2b)


def _knn_idx(x):
    n, f = x.shape
    idx = pl.pallas_call(
        _knn_kernel,
        grid=(n // BR,),
        in_specs=[
            pl.BlockSpec((BR, f), lambda i: (i, 0)),
            pl.BlockSpec((n, f), lambda i: (0, 0)),
        ],
        out_specs=pl.BlockSpec((BR, 128), lambda i: (i, 0)),
        out_shape=jax.ShapeDtypeStruct((n, 128), jnp.int32),
        scratch_shapes=[pltpu.VMEM((BR, n), jnp.float32)],
    )(x, x)
    return idx[:, 1:N_NBRS + 1]


def _sc_gather(x, idx_flat):
    v, d = x.shape
    b = idx_flat.shape[0]
    dp = 128
    xp = jnp.pad(x, ((0, 0), (0, dp - d)))
    info = plsc.get_sparse_core_info()
    nw = info.num_cores * info.num_subcores
    assert b % (8 * nw) == 0
    b_per_w = b // nw
    chunk = 512
    mesh = plsc.VectorSubcoreMesh(core_axis_name="c", subcore_axis_name="s")

    @functools.partial(
        pl.kernel, mesh=mesh,
        out_type=jax.ShapeDtypeStruct((b, dp), jnp.float32),
        scratch_types=[
            pltpu.VMEM((b_per_w,), jnp.int32),
            pltpu.VMEM((chunk, dp), jnp.float32),
            pltpu.SemaphoreType.DMA,
        ],
    )
    def k(table_hbm, idx_hbm, out_hbm, idx_v, rows_v, sem):
        wid = lax.axis_index("s") * info.num_cores + lax.axis_index("c")
        base = wid * b_per_w
        pltpu.sync_copy(idx_hbm.at[pl.ds(base, b_per_w)], idx_v)
        for c in range(b_per_w // chunk):
            pltpu.async_copy(
                table_hbm.at[idx_v.at[pl.ds(c * chunk, chunk)]], rows_v,
                sem).wait()
            pltpu.sync_copy(rows_v, out_hbm.at[pl.ds(base + c * chunk, chunk)])

    return k(xp, idx_flat)[:, :d]


BN = 256


def _proj_kernel(q_ref, o_ref):
    k = N_NBRS
    q0 = q_ref[:, 0 * k:1 * k]
    q1 = q_ref[:, 1 * k:2 * k]
    q2 = q_ref[:, 2 * k:3 * k]
    for a in range(k):
        acc = q0[:, a:a + 1] * q0
        acc = acc + q1[:, a:a + 1] * q1
        acc = acc + q2[:, a:a + 1] * q2
        eye_row = jnp.where(
            jax.lax.broadcasted_iota(jnp.int32, (BN, k), 1) == a, 1.0, 0.0)
        o_ref[:, a * k:(a + 1) * k] = eye_row - acc


def _projectors(q):
    n, k, p = q.shape
    q2 = jnp.swapaxes(q, 1, 2).reshape(n, p * k)
    out = pl.pallas_call(
        _proj_kernel,
        grid=(n // BN,),
        in_specs=[pl.BlockSpec((BN, p * k), lambda i: (i, 0))],
        out_specs=pl.BlockSpec((BN, k * k), lambda i: (i, 0)),
        out_shape=jax.ShapeDtypeStruct((n, k * k), jnp.float32),
    )(q2)
    return out.reshape(n, k, k)


def _eval_basis(theta, d):
    p = d * (d + 1) // 2
    cols = [theta[..., i] for i in range(d)]
    for i in range(d):
        for j in range(i, d):
            cols.append(theta[..., i] * theta[..., j])
    return jnp.stack(cols[:p], axis=-1)


def kernel(x):
    n, f = x.shape
    d = N_COMP
    k = N_NBRS

    idx = _knn_idx(x)

    nbrs = _sc_gather(x, idx.reshape(-1)).reshape(n, k, f)
    centered = nbrs - x[:, None, :]
    cov = jnp.einsum('nkf,nkg->nfg', centered, centered) / (k - 1)
    w, v = jnp.linalg.eigh(cov)
    w_desc = w[:, ::-1]
    v_desc = v[:, :, ::-1]
    valid = (w_desc[:, :d] > 1e-10).astype(x.dtype)
    frames = jnp.swapaxes(v_desc[:, :, :d], 1, 2) * valid[:, :, None]

    theta = jnp.einsum('nkf,ndf->nkd', nbrs, frames)
    B = _eval_basis(theta, d)
    Q, _ = jnp.linalg.qr(B)
    P = jnp.eye(k, dtype=x.dtype)[None, :, :] - jnp.einsum('nkp,nlp->nkl', Q, Q)
    rows = idx[:, :, None]
    cols = idx[:, None, :]
    H = jnp.zeros((n, n), dtype=x.dtype).at[rows, cols].add(P)

    w, v = jnp.linalg.eigh(H)
    order = jnp.argsort(w)[d + 1: 2 * d + 1]
    return v[:, order]
